# Initial kernel scaffold; baseline (speedup 1.0000x reference)
import jax
import jax.numpy as jnp
from jax.experimental import pallas as pl


def kernel(x, W_router, b_router, w1, w2, w3, gumbel_noise):
    raise NotImplementedError("write your pallas kernel here")


# speedup vs baseline: 2.2386x; 2.2386x over previous
import functools

import jax
import jax.numpy as jnp
from jax.experimental import pallas as pl

_INTERPRET = False


def _routing_body(k_sel, x_ref, wr_ref, br_ref, g_ref, w_ref):
    xf = x_ref[...]
    logits = jnp.dot(xf, wr_ref[...], preferred_element_type=jnp.float32)
    logits = logits + br_ref[...] + g_ref[...]
    m = jnp.max(logits, axis=1, keepdims=True)
    e = jnp.exp(logits - m)
    P = e / jnp.sum(e, axis=1, keepdims=True)

    bs = P.shape[0]
    Pt = P.T
    A = jax.lax.bitcast_convert_type(Pt, jnp.int32)
    one = jnp.int32(1)

    T = jnp.zeros((A.shape[0], 1), jnp.int32)
    for bit in range(30, -1, -1):
        cand = T | (one << bit)
        cnt = jnp.sum(jnp.where(A >= cand, 1, 0), axis=1, keepdims=True)
        T = jnp.where(cnt >= k_sel, cand, T)
    gt = A > T
    cnt_gt = jnp.sum(jnp.where(gt, 1, 0), axis=1, keepdims=True)
    r = k_sel - cnt_gt
    eq = A == T
    idxv = jax.lax.broadcasted_iota(jnp.int32, A.shape, 1)
    M = jnp.zeros((A.shape[0], 1), jnp.int32)
    for bit in range(12, -1, -1):
        cand = M | (one << bit)
        cnte = jnp.sum(jnp.where(eq & (idxv < cand), 1, 0), axis=1,
                       keepdims=True)
        M = jnp.where(cnte < r, cand, M)
    include = gt | (eq & (idxv <= M))
    wt = jnp.sum(Pt * include.astype(jnp.float32), axis=0, keepdims=True)
    w_ref[...] = wt.T


def _ffn_body(x_ref, w1_ref, w2_ref, w3_ref, wv_ref, o_ref):
    xb = x_ref[...]
    a = jnp.dot(xb, w1_ref[...], preferred_element_type=jnp.float32)
    b = jnp.dot(xb, w2_ref[...], preferred_element_type=jnp.float32)
    h = a * jax.lax.logistic(a) * b
    o = jnp.dot(h, w3_ref[...], preferred_element_type=jnp.float32)
    o_ref[...] = o * wv_ref[...]


def kernel(x, W_router, b_router, w1, w2, w3, gumbel_noise):
    B, S, H = x.shape
    bs = B * S
    E = W_router.shape[1]
    k_sel = min(bs // E, bs)
    FF = w1.shape[1]
    xf = x.reshape(bs, H)

    wv = pl.pallas_call(
        functools.partial(_routing_body, k_sel),
        out_shape=jax.ShapeDtypeStruct((bs, 1), jnp.float32),
        interpret=_INTERPRET,
    )(xf, W_router, b_router.reshape(1, E), gumbel_noise)

    FFP = ((FF + 127) // 128) * 128
    pad = FFP - FF
    w1p = jnp.pad(w1, ((0, 0), (0, pad)))
    w2p = jnp.pad(w2, ((0, 0), (0, pad)))
    w3p = jnp.pad(w3, ((0, pad), (0, 0)))

    BM = 512
    grid = (bs // BM,)
    y = pl.pallas_call(
        _ffn_body,
        grid=grid,
        in_specs=[
            pl.BlockSpec((BM, H), lambda i: (i, 0)),
            pl.BlockSpec((H, FFP), lambda i: (0, 0)),
            pl.BlockSpec((H, FFP), lambda i: (0, 0)),
            pl.BlockSpec((FFP, H), lambda i: (0, 0)),
            pl.BlockSpec((BM, 1), lambda i: (i, 0)),
        ],
        out_specs=pl.BlockSpec((BM, H), lambda i: (i, 0)),
        out_shape=jax.ShapeDtypeStruct((bs, H), jnp.float32),
        interpret=_INTERPRET,
    )(xf, w1p, w2p, w3p, wv)

    return y.reshape(B, S, H)


# speedup vs baseline: 2.8234x; 1.2613x over previous
import functools

import jax
import jax.numpy as jnp
from jax.experimental import pallas as pl

_INTERPRET = False


def _routing_body(k_sel, x_ref, wr_ref, br_ref, g_ref, w_ref):
    xf = x_ref[...]
    logits = jnp.dot(xf, wr_ref[...], preferred_element_type=jnp.float32)
    logits = logits + br_ref[...] + g_ref[...]
    m = jnp.max(logits, axis=1, keepdims=True)
    e = jnp.exp(logits - m)
    P = e / jnp.sum(e, axis=1, keepdims=True)

    bs = P.shape[0]
    Pt = P.T
    A = jax.lax.bitcast_convert_type(Pt, jnp.int32)
    one = jnp.int32(1)

    T = jnp.zeros((A.shape[0], 1), jnp.int32)
    for bit in range(30, -1, -1):
        cand = T | (one << bit)
        cnt = jnp.sum(jnp.where(A >= cand, 1, 0), axis=1, keepdims=True)
        T = jnp.where(cnt >= k_sel, cand, T)
    gt = A > T
    cnt_gt = jnp.sum(jnp.where(gt, 1, 0), axis=1, keepdims=True)
    r = k_sel - cnt_gt
    eq = A == T
    idxv = jax.lax.broadcasted_iota(jnp.int32, A.shape, 1)
    M = jnp.zeros((A.shape[0], 1), jnp.int32)
    for bit in range(12, -1, -1):
        cand = M | (one << bit)
        cnte = jnp.sum(jnp.where(eq & (idxv < cand), 1, 0), axis=1,
                       keepdims=True)
        M = jnp.where(cnte < r, cand, M)
    include = gt | (eq & (idxv <= M))
    wt = jnp.sum(Pt * include.astype(jnp.float32), axis=0, keepdims=True)
    w_ref[...] = wt.T


def _ffn_body(x_ref, w1_ref, w2_ref, w3_ref, wv_ref, o_ref):
    xb = x_ref[...]
    a = jnp.dot(xb, w1_ref[...], preferred_element_type=jnp.float32)
    b = jnp.dot(xb, w2_ref[...], preferred_element_type=jnp.float32)
    h = a * jax.lax.logistic(a) * b
    o = jnp.dot(h, w3_ref[...], preferred_element_type=jnp.float32)
    o_ref[...] = o * wv_ref[...]


def kernel(x, W_router, b_router, w1, w2, w3, gumbel_noise):
    B, S, H = x.shape
    bs = B * S
    E = W_router.shape[1]
    k_sel = min(bs // E, bs)
    FF = w1.shape[1]
    xf = x.reshape(bs, H)

    wv = pl.pallas_call(
        functools.partial(_routing_body, k_sel),
        out_shape=jax.ShapeDtypeStruct((bs, 1), jnp.float32),
        interpret=_INTERPRET,
    )(xf, W_router, b_router.reshape(1, E), gumbel_noise)

    BM = 512
    grid = (bs // BM,)
    y = pl.pallas_call(
        _ffn_body,
        grid=grid,
        in_specs=[
            pl.BlockSpec((BM, H), lambda i: (i, 0)),
            pl.BlockSpec((H, FF), lambda i: (0, 0)),
            pl.BlockSpec((H, FF), lambda i: (0, 0)),
            pl.BlockSpec((FF, H), lambda i: (0, 0)),
            pl.BlockSpec((BM, 1), lambda i: (i, 0)),
        ],
        out_specs=pl.BlockSpec((BM, H), lambda i: (i, 0)),
        out_shape=jax.ShapeDtypeStruct((bs, H), jnp.float32),
        interpret=_INTERPRET,
    )(xf, w1, w2, w3, wv)

    return y.reshape(B, S, H)
